# baseline (speedup 1.0000x reference)
import functools
import jax
import jax.numpy as jnp
from jax import lax
from jax.experimental import pallas as pl
from jax.experimental.pallas import tpu as pltpu
from jax.experimental.pallas import tpu_sc as plsc

_N = 2048
_R = 512
_NB = _N // _R
_M = 3 * _NB
_NBUF = 4
_D = 2


def _stream_body(x_ref, b1_ref, b2_ref, b3_ref,
                 w1_hbm, w2_hbm, w3_hbm,
                 o1_hbm, o2_hbm, o3_hbm, out_ref, h1_ref, h2_ref,
                 bufs, load_sems, store_sems):
    w_hbms = [w1_hbm, w2_hbm, w3_hbm]
    o_hbms = [o1_hbm, o2_hbm, o3_hbm]
    b_refs = [b1_ref, b2_ref, b3_ref]
    x_srcs = [x_ref, h1_ref, h2_ref]
    a_dsts = [h1_ref, h2_ref, out_ref]

    def load(k):
        l, b = divmod(k, _NB)
        s = k % _NBUF
        cp = pltpu.make_async_copy(
            w_hbms[l].at[pl.ds(b * _R, _R), :], bufs.at[s], load_sems.at[s])
        cp.start()
        return cp

    def store(k):
        l, b = divmod(k, _NB)
        s = k % _NBUF
        cp = pltpu.make_async_copy(
            bufs.at[s], o_hbms[l].at[pl.ds(b * _R, _R), :], store_sems.at[s])
        cp.start()
        return cp

    loads, stores = {}, {}
    for k in range(min(_D, _M)):
        loads[k] = load(k)

    for k in range(_M):
        l, b = divmod(k, _NB)
        s = k % _NBUF
        kd = k + _D
        if kd < _M:
            if kd >= _NBUF:
                stores[kd - _NBUF].wait()
            loads[kd] = load(kd)
        loads[k].wait()

        x = x_srcs[l][...]
        w = bufs[s]
        y = jnp.sum(w * x, axis=1)
        a = jnp.maximum(y + b_refs[l][pl.ds(b * _R, _R)], 0.0)
        a_dsts[l][0:1, pl.ds(b * _R, _R)] = a[None, :]

        stores[k] = store(k)

    for k in range(max(0, _M - _NBUF), _M):
        stores[k].wait()


def _stream(x, W1, b1, W2, b2, W3, b3):
    f32 = jnp.float32
    return pl.pallas_call(
        _stream_body,
        in_specs=[pl.BlockSpec(memory_space=pltpu.VMEM)] * 4
        + [pl.BlockSpec(memory_space=pl.ANY)] * 3,
        out_specs=[pl.BlockSpec(memory_space=pl.ANY)] * 3
        + [pl.BlockSpec(memory_space=pltpu.VMEM)] * 3,
        out_shape=[jax.ShapeDtypeStruct((_N, _N), f32)] * 3
        + [jax.ShapeDtypeStruct((1, _N), f32)] * 3,
        scratch_shapes=[
            pltpu.VMEM((_NBUF, _R, _N), f32),
            pltpu.SemaphoreType.DMA((_NBUF,)),
            pltpu.SemaphoreType.DMA((_NBUF,)),
        ],
    )(x, b1, b2, b3, W1, W2, W3)


def _sc_meta_body(x_hbm, h1_hbm, h2_hbm, out_hbm, meta_hbm,
                  w1_hbm, w2_hbm, w3_hbm, p_hbm,
                  acts_v, ws_v, mv, pv):
    wid = lax.axis_index("s") * 2 + lax.axis_index("c")

    @pl.when(wid == 0)
    def _():
        for i, src in enumerate((x_hbm, h1_hbm, h2_hbm, out_hbm)):
            pltpu.sync_copy(src.at[pl.ds(0, 16)], acts_v.at[i])
        for i, src in enumerate((w1_hbm, w2_hbm, w3_hbm)):
            pltpu.sync_copy(src.at[pl.ds(0, 16)], ws_v.at[i])
        pltpu.sync_copy(meta_hbm, mv)

        lane = lax.iota(jnp.int32, 16)
        izero = lane * 0
        inp = plsc.load_gather(acts_v, [lane, izero])
        aout = plsc.load_gather(acts_v, [jnp.minimum(lane + 1, 3), izero])
        w00 = plsc.load_gather(ws_v, [jnp.minimum(lane, 2), izero])
        m0 = plsc.load_gather(mv, [izero])
        m1 = plsc.load_gather(mv, [izero + 1])
        m2 = plsc.load_gather(mv, [izero + 2])
        mb = plsc.load_gather(mv, [izero + 3])
        p = m0 * inp + m1 * w00 + m2 * aout + mb
        pv[...] = p
        pltpu.sync_copy(pv, p_hbm)


def _sc_meta(xf, h1f, h2f, outf, meta_vec, W1f, W2f, W3f):
    mesh = plsc.VectorSubcoreMesh(core_axis_name="c", subcore_axis_name="s")
    f32 = jnp.float32
    fn = functools.partial(
        pl.kernel,
        out_type=jax.ShapeDtypeStruct((16,), f32),
        mesh=mesh,
        compiler_params=pltpu.CompilerParams(needs_layout_passes=False),
        scratch_types=[
            pltpu.VMEM((4, 16), f32),
            pltpu.VMEM((3, 16), f32),
            pltpu.VMEM((16,), f32),
            pltpu.VMEM((16,), f32),
        ],
    )(_sc_meta_body)
    return fn(xf, h1f, h2f, outf, meta_vec, W1f, W2f, W3f)


def _patch_body(p_ref, w1_ref, w2_ref, w3_ref, o1_ref, o2_ref, o3_ref):
    rows = jax.lax.broadcasted_iota(jnp.int32, (8, 128), 0)
    cols = jax.lax.broadcasted_iota(jnp.int32, (8, 128), 1)
    m = (rows == 0) & (cols == 0)
    for i, (w_ref, o_ref) in enumerate(
            ((w1_ref, o1_ref), (w2_ref, o2_ref), (w3_ref, o3_ref))):
        o_ref[...] = jnp.where(m, p_ref[i], w_ref[...])


def _patch(p, nw1, nw2, nw3):
    f32 = jnp.float32
    blk = pl.BlockSpec((8, 128), lambda i: (0, 0))
    return pl.pallas_call(
        _patch_body,
        grid=(1,),
        in_specs=[pl.BlockSpec(memory_space=pltpu.VMEM), blk, blk, blk],
        out_specs=[blk, blk, blk],
        out_shape=[jax.ShapeDtypeStruct((_N, _N), f32)] * 3,
        input_output_aliases={1: 0, 2: 1, 3: 2},
    )(p, nw1, nw2, nw3)


def kernel(x, W1, b1, W2, b2, W3, b3, meta_W, meta_b):
    nw1, nw2, nw3, out, h1, h2 = _stream(x, W1, b1, W2, b2, W3, b3)
    meta_vec = jnp.concatenate(
        [meta_W[0], meta_b, jnp.zeros((12,), jnp.float32)])
    p = _sc_meta(x.reshape(-1), h1.reshape(-1), h2.reshape(-1),
                 out.reshape(-1), meta_vec,
                 W1.reshape(-1), W2.reshape(-1), W3.reshape(-1))
    nw1, nw2, nw3 = _patch(p, nw1, nw2, nw3)
    return out, nw1, nw2, nw3
